# baseline (speedup 1.0000x reference)
import jax
import jax.numpy as jnp
from jax import lax
from jax.experimental import pallas as pl
from jax.experimental.pallas import tpu as pltpu

T = 4096
D = 1024
DFF = 2048
E = 8
NE = E + 1
FEAT = 8
INNER = 1024
CHESS_BIAS = 2.0

DPAD = 1152
XTRA = DPAD - D
BLK = 512
NB_STD = 23
NB_CHS = 8
NB = NB_STD + NB_CHS
STD_ROWS = NB_STD * BLK
PADN = NB * BLK
P = T * 2

_NEG = -1e30


_BT = 512
_NBK = T // _BT


def _rmeta_body(x_ref, wr_ref, w_out, pos_out, meta_out,
                pref_s, idx_s, w_s, off_s, bs_s):
    i = pl.program_id(0)

    @pl.when(i == 0)
    def _init():
        off_s[...] = jnp.zeros((1, 128), jnp.float32)

    @pl.when(i < _NBK)
    def _pass1():
        x = x_ref[...]
        lg = jax.lax.dot_general(x, wr_ref[...], (((1,), (0,)), ((), ())),
                                 preferred_element_type=jnp.float32)
        lane = lax.broadcasted_iota(jnp.int32, (_BT, 128), 1)
        lg = jnp.where(lane >= NE, _NEG, lg)
        m1 = jnp.max(lg, axis=1, keepdims=True)
        i1 = jnp.min(jnp.where(lg == m1, lane, 127), axis=1, keepdims=True)
        lg2 = jnp.where(lane == i1, _NEG, lg)
        m2 = jnp.max(lg2, axis=1, keepdims=True)
        i2 = jnp.min(jnp.where(lg2 == m2, lane, 127), axis=1, keepdims=True)
        w1 = 1.0 / (1.0 + jnp.exp(m2 - m1))
        w2 = 1.0 - w1
        onep = jnp.ones((_BT, 128), jnp.float32)
        zerop = jnp.zeros((_BT, 128), jnp.float32)
        c = (jnp.where(lane == i1, onep, zerop)
             + jnp.where(lane == i2, onep, zerop))
        r0 = lax.broadcasted_iota(jnp.int32, (_BT, _BT), 0)
        r1 = lax.broadcasted_iota(jnp.int32, (_BT, _BT), 1)
        ltri = (r1 < r0).astype(jnp.float32)
        prefb = jax.lax.dot_general(ltri, c, (((1,), (0,)), ((), ())),
                                    preferred_element_type=jnp.float32)
        prefb = prefb + off_s[...]
        sl = pl.ds(i * _BT, _BT)
        pref_s[sl, :] = prefb
        idx_s[sl, :] = jnp.where(lane == 0, i1, jnp.where(lane == 1, i2, 0))
        w_s[sl, :] = jnp.where(lane == 0, w1, jnp.where(lane == 1, w2, 0.0))
        off_s[...] = off_s[...] + jnp.sum(c, axis=0, keepdims=True)

        @pl.when(i == _NBK - 1)
        def _epilogue():
            counts = off_s[...]
            lr = lax.broadcasted_iota(jnp.int32, (1, 128), 1)
            bpe = jnp.where(lr < E,
                            jnp.floor((counts + (BLK - 1)) / BLK), 0.0)
            u0 = lax.broadcasted_iota(jnp.int32, (128, 128), 0)
            u1 = lax.broadcasted_iota(jnp.int32, (128, 128), 1)
            utri = (u0 < u1).astype(jnp.float32)
            bsb = jax.lax.dot_general(bpe, utri, (((1,), (0,)), ((), ())),
                                      preferred_element_type=jnp.float32)
            bs_rows = jnp.where(lr == E, float(STD_ROWS), bsb * BLK)
            bs_s[...] = bs_rows
            n_std = jnp.sum(jnp.where(lr == E, bsb, 0.0), axis=1,
                            keepdims=True)
            n_chb = jnp.sum(jnp.where(lr == E,
                                      jnp.floor((counts + (BLK - 1)) / BLK),
                                      0.0), axis=1, keepdims=True)
            one = jnp.ones((128, 128), jnp.float32)
            zero = jnp.zeros((128, 128), jnp.float32)
            bsb_b = jnp.broadcast_to(bsb, (128, 128))
            emask = jnp.logical_and(u1 >= 1, u1 <= E)
            u0f = u0.astype(jnp.float32)
            ge_f = jnp.where(jnp.logical_and(u0f >= bsb_b, emask), one, zero)
            bec = jnp.minimum(jnp.sum(ge_f, axis=1, keepdims=True),
                              float(E - 1))
            bec_b = jnp.broadcast_to(bec, (128, 128))
            n_std_b = jnp.broadcast_to(n_std, (128, 128))
            n_chb_b = jnp.broadcast_to(n_chb, (128, 128))
            valid_f = jnp.where(
                u0f < float(NB_STD),
                jnp.where(u0f < n_std_b, one, zero),
                jnp.where(u0f - float(NB_STD) < n_chb_b, one, zero))
            meta_f = jnp.where(u1 == 0, bec_b,
                               jnp.where(u1 == 1, valid_f, zero))
            meta_out[...] = meta_f.astype(jnp.int32)

    @pl.when(i >= _NBK)
    def _pass2():
        sl = pl.ds((i - _NBK) * _BT, _BT)
        prefb = pref_s[sl, :]
        idxb = idx_s[sl, :]
        lane = lax.broadcasted_iota(jnp.int32, (_BT, 128), 1)
        tot = prefb + bs_s[...]
        i1 = idxb[:, 0:1]
        i2 = idxb[:, 1:2]
        d1 = jnp.sum(jnp.where(lane == i1, tot, 0.0), axis=1, keepdims=True)
        d2 = jnp.sum(jnp.where(lane == i2, tot, 0.0), axis=1, keepdims=True)
        pos_f = jnp.where(lane == 0, d1, jnp.where(lane == 1, d2, 0.0))
        pos_out[...] = pos_f.astype(jnp.int32)
        w_out[...] = w_s[sl, :]


def _run_rmeta(x_aug, wr_aug):
    return pl.pallas_call(
        _rmeta_body,
        grid=(2 * _NBK,),
        in_specs=[
            pl.BlockSpec((_BT, DPAD), lambda i: (jnp.minimum(i, _NBK - 1), 0)),
            pl.BlockSpec((DPAD, 128), lambda i: (0, 0)),
        ],
        out_specs=[
            pl.BlockSpec((_BT, 128), lambda i: (jnp.maximum(i - _NBK, 0), 0)),
            pl.BlockSpec((_BT, 128), lambda i: (jnp.maximum(i - _NBK, 0), 0)),
            pl.BlockSpec((128, 128), lambda i: (0, 0)),
        ],
        out_shape=[
            jax.ShapeDtypeStruct((T, 128), jnp.float32),
            jax.ShapeDtypeStruct((T, 128), jnp.int32),
            jax.ShapeDtypeStruct((128, 128), jnp.int32),
        ],
        scratch_shapes=[
            pltpu.VMEM((T, 128), jnp.float32),
            pltpu.VMEM((T, 128), jnp.int32),
            pltpu.VMEM((T, 128), jnp.float32),
            pltpu.VMEM((1, 128), jnp.float32),
            pltpu.VMEM((1, 128), jnp.float32),
        ],
    )(x_aug, wr_aug)


def _moe_body(be_ref, bvalid_ref, x_ref, w1_ref, w2_ref, b1_ref, b2_ref,
              wc1_ref, bc1_ref, wc2_ref, bc2_ref, wrow_ref, y_ref):
    b = pl.program_id(0)

    @pl.when(bvalid_ref[b] != 0)
    def _():
        @pl.when(b < NB_STD)
        def _std():
            x = x_ref[...][:, :D]
            h = jax.lax.dot_general(x, w1_ref[0], (((1,), (0,)), ((), ())),
                                    preferred_element_type=jnp.float32)
            h = jax.nn.gelu(h + b1_ref[0, 0][None, :])
            y = jax.lax.dot_general(h, w2_ref[0], (((1,), (0,)), ((), ())),
                                    preferred_element_type=jnp.float32)
            y = y + b2_ref[0, 0][None, :]
            y_ref[...] = y * wrow_ref[...][:, 0:1]

        @pl.when(b >= NB_STD)
        def _chess():
            x = x_ref[...]
            h = jax.lax.dot_general(x[:, :D], wc1_ref[:D], (((1,), (0,)), ((), ())),
                                    preferred_element_type=jnp.float32)
            h = h + jax.lax.dot_general(x[:, D:], wc1_ref[D:],
                                        (((1,), (0,)), ((), ())),
                                        preferred_element_type=jnp.float32)
            h = jax.nn.gelu(h + bc1_ref[0][None, :])
            y = jax.lax.dot_general(h, wc2_ref[...], (((1,), (0,)), ((), ())),
                                    preferred_element_type=jnp.float32)
            y = y + bc2_ref[0][None, :]
            y_ref[...] = y * wrow_ref[...][:, 0:1]


def _run_moe(x_sorted, W1, W2, b1, b2, wc1p, bc1, Wc2, bc2, w_rows,
             block_expert, block_valid):
    grid_spec = pltpu.PrefetchScalarGridSpec(
        num_scalar_prefetch=2,
        grid=(NB,),
        in_specs=[
            pl.BlockSpec((BLK, DPAD), lambda b, be, bv: (b, 0)),
            pl.BlockSpec((1, D, DFF), lambda b, be, bv: (be[b], 0, 0)),
            pl.BlockSpec((1, DFF, D), lambda b, be, bv: (be[b], 0, 0)),
            pl.BlockSpec((1, 1, DFF), lambda b, be, bv: (be[b], 0, 0)),
            pl.BlockSpec((1, 1, D), lambda b, be, bv: (be[b], 0, 0)),
            pl.BlockSpec((DPAD, INNER), lambda b, be, bv: (0, 0)),
            pl.BlockSpec((1, INNER), lambda b, be, bv: (0, 0)),
            pl.BlockSpec((INNER, D), lambda b, be, bv: (0, 0)),
            pl.BlockSpec((1, D), lambda b, be, bv: (0, 0)),
            pl.BlockSpec((BLK, 8), lambda b, be, bv: (b, 0)),
        ],
        out_specs=pl.BlockSpec((BLK, D), lambda b, be, bv: (b, 0)),
    )
    return pl.pallas_call(
        _moe_body,
        grid_spec=grid_spec,
        out_shape=jax.ShapeDtypeStruct((PADN, D), jnp.float32),
    )(block_expert, block_valid, x_sorted, W1, W2, b1, b2,
      wc1p, bc1, Wc2, bc2, w_rows)


def kernel(hidden_states, chess_eval, mask_is_chess, router_W,
           W1, b1, W2, b2, Wc1, bc1, Wc2, bc2):
    f32 = jnp.float32
    i32 = jnp.int32
    mask_f = mask_is_chess.astype(f32)
    feats = chess_eval * mask_f[:, None]

    x_aug = jnp.zeros((T, DPAD), f32)
    x_aug = x_aug.at[:, :D].set(hidden_states)
    x_aug = x_aug.at[:, D:D + FEAT].set(feats)
    x_aug = x_aug.at[:, D + FEAT].set(mask_f)

    wr_aug = jnp.zeros((DPAD, 128), f32)
    wr_aug = wr_aug.at[:D, :NE].set(router_W)
    wr_aug = wr_aug.at[D + FEAT, E].set(CHESS_BIAS)

    wc1p = jnp.zeros((DPAD, INNER), f32).at[:D + FEAT].set(Wc1)
    b1r = b1.reshape(E, 1, DFF)
    b2r = b2.reshape(E, 1, D)
    bc1r = bc1.reshape(1, INNER)
    bc2r = bc2.reshape(1, D)

    w128, pos128, meta = _run_rmeta(x_aug, wr_aug)
    pos = pos128[:, :2]
    dest = pos.reshape(P)
    w_flat = w128[:, :2].reshape(P)
    block_expert = meta[:NB, 0]
    block_valid = meta[:NB, 1]
    pay = jnp.stack([(jnp.arange(P, dtype=i32) // 2).astype(f32), w_flat],
                    axis=1)
    buf = jnp.zeros((PADN, 2), f32).at[dest].set(pay)
    tok_of_row = buf[:, 0].astype(i32)
    w_of_row = buf[:, 1]

    x_sorted = jnp.take(x_aug, tok_of_row, axis=0)

    w_rows = jnp.broadcast_to(w_of_row[:, None], (PADN, 8))
    y = _run_moe(x_sorted, W1, W2, b1r, b2r, wc1p, bc1r, Wc2, bc2r,
                 w_rows, block_expert, block_valid)

    out = jnp.take(y, dest, axis=0).reshape(T, 2, D).sum(axis=1)
    return out


# speedup vs baseline: 1.1470x; 1.1470x over previous
import jax
import jax.numpy as jnp
from jax import lax
from jax.experimental import pallas as pl
from jax.experimental.pallas import tpu as pltpu

T = 4096
D = 1024
DFF = 2048
E = 8
NE = E + 1
FEAT = 8
INNER = 1024
CHESS_BIAS = 2.0

DPAD = 1152
XTRA = DPAD - D
BLK = 512
NB_STD = 23
NB_CHS = 8
NB = NB_STD + NB_CHS
STD_ROWS = NB_STD * BLK
PADN = NB * BLK
P = T * 2

_NEG = -1e30


_BT = 512
_NBK = T // _BT


def _rmeta_body(x_ref, wr_ref, w_out, pos_out, meta_out,
                pref_s, idx_s, w_s, off_s, bs_s):
    i = pl.program_id(0)

    @pl.when(i == 0)
    def _init():
        off_s[...] = jnp.zeros((1, 128), jnp.float32)

    @pl.when(i < _NBK)
    def _pass1():
        x = x_ref[...]
        lg = jax.lax.dot_general(x, wr_ref[...], (((1,), (0,)), ((), ())),
                                 preferred_element_type=jnp.float32)
        lane = lax.broadcasted_iota(jnp.int32, (_BT, 128), 1)
        lg = jnp.where(lane >= NE, _NEG, lg)
        m1 = jnp.max(lg, axis=1, keepdims=True)
        i1 = jnp.min(jnp.where(lg == m1, lane, 127), axis=1, keepdims=True)
        lg2 = jnp.where(lane == i1, _NEG, lg)
        m2 = jnp.max(lg2, axis=1, keepdims=True)
        i2 = jnp.min(jnp.where(lg2 == m2, lane, 127), axis=1, keepdims=True)
        w1 = 1.0 / (1.0 + jnp.exp(m2 - m1))
        w2 = 1.0 - w1
        onep = jnp.ones((_BT, 128), jnp.float32)
        zerop = jnp.zeros((_BT, 128), jnp.float32)
        c = (jnp.where(lane == i1, onep, zerop)
             + jnp.where(lane == i2, onep, zerop))
        r0 = lax.broadcasted_iota(jnp.int32, (_BT, _BT), 0)
        r1 = lax.broadcasted_iota(jnp.int32, (_BT, _BT), 1)
        ltri = (r1 < r0).astype(jnp.float32)
        prefb = jax.lax.dot_general(ltri, c, (((1,), (0,)), ((), ())),
                                    preferred_element_type=jnp.float32)
        prefb = prefb + off_s[...]
        sl = pl.ds(i * _BT, _BT)
        pref_s[sl, :] = prefb
        idx_s[sl, :] = jnp.where(lane == 0, i1, jnp.where(lane == 1, i2, 0))
        w_s[sl, :] = jnp.where(lane == 0, w1, jnp.where(lane == 1, w2, 0.0))
        off_s[...] = off_s[...] + jnp.sum(c, axis=0, keepdims=True)

        @pl.when(i == _NBK - 1)
        def _epilogue():
            counts = off_s[...]
            lr = lax.broadcasted_iota(jnp.int32, (1, 128), 1)
            bpe = jnp.where(lr < E,
                            jnp.floor((counts + (BLK - 1)) / BLK), 0.0)
            u0 = lax.broadcasted_iota(jnp.int32, (128, 128), 0)
            u1 = lax.broadcasted_iota(jnp.int32, (128, 128), 1)
            utri = (u0 < u1).astype(jnp.float32)
            bsb = jax.lax.dot_general(bpe, utri, (((1,), (0,)), ((), ())),
                                      preferred_element_type=jnp.float32)
            bs_rows = jnp.where(lr == E, float(STD_ROWS), bsb * BLK)
            bs_s[...] = bs_rows
            n_std = jnp.sum(jnp.where(lr == E, bsb, 0.0), axis=1,
                            keepdims=True)
            n_chb = jnp.sum(jnp.where(lr == E,
                                      jnp.floor((counts + (BLK - 1)) / BLK),
                                      0.0), axis=1, keepdims=True)
            one = jnp.ones((128, 128), jnp.float32)
            zero = jnp.zeros((128, 128), jnp.float32)
            bsb_b = jnp.broadcast_to(bsb, (128, 128))
            emask = jnp.logical_and(u1 >= 1, u1 <= E)
            u0f = u0.astype(jnp.float32)
            ge_f = jnp.where(jnp.logical_and(u0f >= bsb_b, emask), one, zero)
            bec = jnp.minimum(jnp.sum(ge_f, axis=1, keepdims=True),
                              float(E - 1))
            bec_b = jnp.broadcast_to(bec, (128, 128))
            n_std_b = jnp.broadcast_to(n_std, (128, 128))
            n_chb_b = jnp.broadcast_to(n_chb, (128, 128))
            valid_f = jnp.where(
                u0f < float(NB_STD),
                jnp.where(u0f < n_std_b, one, zero),
                jnp.where(u0f - float(NB_STD) < n_chb_b, one, zero))
            meta_f = jnp.where(u1 == 0, bec_b,
                               jnp.where(u1 == 1, valid_f, zero))
            meta_out[...] = meta_f.astype(jnp.int32)

    @pl.when(i >= _NBK)
    def _pass2():
        sl = pl.ds((i - _NBK) * _BT, _BT)
        prefb = pref_s[sl, :]
        idxb = idx_s[sl, :]
        lane = lax.broadcasted_iota(jnp.int32, (_BT, 128), 1)
        tot = prefb + bs_s[...]
        i1 = idxb[:, 0:1]
        i2 = idxb[:, 1:2]
        d1 = jnp.sum(jnp.where(lane == i1, tot, 0.0), axis=1, keepdims=True)
        d2 = jnp.sum(jnp.where(lane == i2, tot, 0.0), axis=1, keepdims=True)
        pos_f = jnp.where(lane == 0, d1, jnp.where(lane == 1, d2, 0.0))
        pos_out[...] = pos_f.astype(jnp.int32)
        w_out[...] = w_s[sl, :]


def _run_rmeta(x_aug, wr_aug):
    return pl.pallas_call(
        _rmeta_body,
        grid=(2 * _NBK,),
        in_specs=[
            pl.BlockSpec((_BT, DPAD), lambda i: (jnp.minimum(i, _NBK - 1), 0)),
            pl.BlockSpec((DPAD, 128), lambda i: (0, 0)),
        ],
        out_specs=[
            pl.BlockSpec((_BT, 128), lambda i: (jnp.maximum(i - _NBK, 0), 0)),
            pl.BlockSpec((_BT, 128), lambda i: (jnp.maximum(i - _NBK, 0), 0)),
            pl.BlockSpec((128, 128), lambda i: (0, 0)),
        ],
        out_shape=[
            jax.ShapeDtypeStruct((T, 128), jnp.float32),
            jax.ShapeDtypeStruct((T, 128), jnp.int32),
            jax.ShapeDtypeStruct((128, 128), jnp.int32),
        ],
        scratch_shapes=[
            pltpu.VMEM((T, 128), jnp.float32),
            pltpu.VMEM((T, 128), jnp.int32),
            pltpu.VMEM((T, 128), jnp.float32),
            pltpu.VMEM((1, 128), jnp.float32),
            pltpu.VMEM((1, 128), jnp.float32),
        ],
    )(x_aug, wr_aug)


def _moe_body(be_ref, bvalid_ref, x_ref, w1_ref, w2_ref, b1_ref, b2_ref,
              wc1_ref, bc1_ref, wc2_ref, bc2_ref, wrow_ref, y_ref):
    b = pl.program_id(0)

    @pl.when(bvalid_ref[b] != 0)
    def _():
        @pl.when(b < NB_STD)
        def _std():
            x = x_ref[...][:, :D]
            h = jax.lax.dot_general(x, w1_ref[0], (((1,), (0,)), ((), ())),
                                    preferred_element_type=jnp.float32)
            h = jax.nn.gelu(h + b1_ref[0, 0][None, :])
            y = jax.lax.dot_general(h, w2_ref[0], (((1,), (0,)), ((), ())),
                                    preferred_element_type=jnp.float32)
            y = y + b2_ref[0, 0][None, :]
            y_ref[...] = y * wrow_ref[...][:, 0:1]

        @pl.when(b >= NB_STD)
        def _chess():
            x = x_ref[...]
            h = jax.lax.dot_general(x[:, :D], wc1_ref[:D], (((1,), (0,)), ((), ())),
                                    preferred_element_type=jnp.float32)
            h = h + jax.lax.dot_general(x[:, D:], wc1_ref[D:],
                                        (((1,), (0,)), ((), ())),
                                        preferred_element_type=jnp.float32)
            h = jax.nn.gelu(h + bc1_ref[0][None, :])
            y = jax.lax.dot_general(h, wc2_ref[...], (((1,), (0,)), ((), ())),
                                    preferred_element_type=jnp.float32)
            y = y + bc2_ref[0][None, :]
            y_ref[...] = y * wrow_ref[...][:, 0:1]


def _run_moe(x_sorted, W1, W2, b1, b2, wc1p, bc1, Wc2, bc2, w_rows,
             block_expert, block_valid):
    grid_spec = pltpu.PrefetchScalarGridSpec(
        num_scalar_prefetch=2,
        grid=(NB,),
        in_specs=[
            pl.BlockSpec((BLK, DPAD), lambda b, be, bv: (b, 0)),
            pl.BlockSpec((1, D, DFF), lambda b, be, bv: (be[b], 0, 0)),
            pl.BlockSpec((1, DFF, D), lambda b, be, bv: (be[b], 0, 0)),
            pl.BlockSpec((1, 1, DFF), lambda b, be, bv: (be[b], 0, 0)),
            pl.BlockSpec((1, 1, D), lambda b, be, bv: (be[b], 0, 0)),
            pl.BlockSpec((DPAD, INNER), lambda b, be, bv: (0, 0)),
            pl.BlockSpec((1, INNER), lambda b, be, bv: (0, 0)),
            pl.BlockSpec((INNER, D), lambda b, be, bv: (0, 0)),
            pl.BlockSpec((1, D), lambda b, be, bv: (0, 0)),
            pl.BlockSpec((BLK, 8), lambda b, be, bv: (b, 0)),
        ],
        out_specs=pl.BlockSpec((BLK, D), lambda b, be, bv: (b, 0)),
    )
    return pl.pallas_call(
        _moe_body,
        grid_spec=grid_spec,
        out_shape=jax.ShapeDtypeStruct((PADN, D), jnp.float32),
    )(block_expert, block_valid, x_sorted, W1, W2, b1, b2,
      wc1p, bc1, Wc2, bc2, w_rows)


def kernel(hidden_states, chess_eval, mask_is_chess, router_W,
           W1, b1, W2, b2, Wc1, bc1, Wc2, bc2):
    f32 = jnp.float32
    i32 = jnp.int32
    mask_f = mask_is_chess.astype(f32)
    feats = chess_eval * mask_f[:, None]

    x_aug = jnp.zeros((T, DPAD), f32)
    x_aug = x_aug.at[:, :D].set(hidden_states)
    x_aug = x_aug.at[:, D:D + FEAT].set(feats)
    x_aug = x_aug.at[:, D + FEAT].set(mask_f)

    wr_aug = jnp.zeros((DPAD, 128), f32)
    wr_aug = wr_aug.at[:D, :NE].set(router_W)
    wr_aug = wr_aug.at[D + FEAT, E].set(CHESS_BIAS)

    wc1p = jnp.zeros((DPAD, INNER), f32).at[:D + FEAT].set(Wc1)
    b1r = b1.reshape(E, 1, DFF)
    b2r = b2.reshape(E, 1, D)
    bc1r = bc1.reshape(1, INNER)
    bc2r = bc2.reshape(1, D)

    w128, pos128, meta = _run_rmeta(x_aug, wr_aug)
    pos = pos128[:, :2]
    dest = pos.reshape(P)
    w_flat = w128[:, :2].reshape(P)
    block_expert = meta[:NB, 0]
    block_valid = meta[:NB, 1]
    pay = jnp.stack([(jnp.arange(P, dtype=i32) // 2).astype(f32), w_flat],
                    axis=1)
    buf = jnp.zeros((PADN, 2), f32).at[dest].set(pay)
    tok_of_row = buf[:, 0].astype(i32)
    w_of_row = buf[:, 1]

    x_sorted = jnp.take(x_aug, tok_of_row, axis=0)

    w_rows = jnp.broadcast_to(w_of_row[:, None], (PADN, 8))
    y = _run_moe(x_sorted, W1, W2, b1r, b2r, wc1p, bc1r, Wc2, bc2r,
                 w_rows, block_expert, block_valid)

    out = jnp.take(y, pos[:, 0], axis=0) + jnp.take(y, pos[:, 1], axis=0)
    return out
